# baseline (speedup 1.0000x reference)
import functools

import jax
import jax.numpy as jnp
from jax import lax
from jax.experimental import pallas as pl
from jax.experimental.pallas import tpu as pltpu
from jax.experimental.pallas import tpu_sc as plsc

N = 10000
E = 320000
D = 128
DW = 128
N_PAD = 10240
POWER1 = 8
NC = 2
NS = 16
EPT = E // NS
K = 128
SB = 16
NSB = (EPT + SB * K - 1) // (SB * K)
NCH = NSB * SB
EPT_PAD = NCH * K
NPT = N_PAD // NS
UC = 16
NUP = NPT // UC

_mesh = plsc.VectorSubcoreMesh(core_axis_name="c", subcore_axis_name="s")
_sc_params = pltpu.CompilerParams(needs_layout_passes=False)


def _prep_body(row_hbm, col_hbm, deg_out, rowp_out,
               eraw, hist, tmp640, dtmp, estage, sh_hist):
    c_id = lax.axis_index("c")
    sid = lax.axis_index("s")

    @pl.when(c_id == 0)
    def _():
        pltpu.sync_copy(col_hbm.at[sid], eraw)

        def zb(k, _):
            hist[pl.ds(k * 16, 16)] = jnp.zeros((16,), jnp.float32)
            return 0
        lax.fori_loop(0, N_PAD // 16, zb, 0)

        ones = jnp.ones((16,), jnp.float32)

        def hb(k, _):
            idx = eraw[pl.ds(k * 16, 16)]
            plsc.addupdate_scatter(hist, [idx], ones)
            return 0
        lax.fori_loop(0, EPT // 16, hb, 0)

        pltpu.sync_copy(hist, sh_hist.at[sid])
        plsc.subcore_barrier()
        r0 = sid * NPT
        pltpu.sync_copy(sh_hist.at[0, pl.ds(r0, NPT)], dtmp)

        def cb(t, _):
            pltpu.sync_copy(sh_hist.at[t, pl.ds(r0, NPT)], tmp640)

            def ab(k, _):
                dtmp[pl.ds(k * 16, 16)] = (dtmp[pl.ds(k * 16, 16)]
                                           + tmp640[pl.ds(k * 16, 16)])
                return 0
            lax.fori_loop(0, NPT // 16, ab, 0)
            return 0
        lax.fori_loop(1, NS, cb, 0)
        pltpu.sync_copy(dtmp, deg_out.at[pl.ds(r0, NPT)])

    @pl.when(c_id == 1)
    def _():
        def mt(t, m):
            pltpu.sync_copy(row_hbm.at[t], eraw)

            def mb(k, mm):
                return jnp.minimum(mm, eraw[pl.ds(k * 16, 16)])
            return lax.fori_loop(0, EPT // 16, mb, m)
        m = lax.fori_loop(0, NS, mt,
                          jnp.full((16,), 2**31 - 1, jnp.int32))
        rminf = jnp.min(m.astype(jnp.float32))
        rminv = jnp.broadcast_to(rminf, (16,)).astype(jnp.int32)

        pltpu.sync_copy(row_hbm.at[sid], eraw)

        def rb(k, _):
            estage[pl.ds(k * 16, 16)] = eraw[pl.ds(k * 16, 16)] - rminv
            return 0
        lax.fori_loop(0, EPT // 16, rb, 0)
        pltpu.sync_copy(estage, rowp_out.at[sid])


_prep_scratch = [
        pltpu.VMEM((EPT,), jnp.int32),
        pltpu.VMEM((N_PAD,), jnp.float32),
        pltpu.VMEM((NPT,), jnp.float32),
        pltpu.VMEM((NPT,), jnp.float32),
        pltpu.VMEM((EPT,), jnp.int32),
        pltpu.VMEM_SHARED((NS, N_PAD), jnp.float32),
]

_prep = functools.partial(
    pl.kernel, _prep_body,
    out_type=(jax.ShapeDtypeStruct((N_PAD,), jnp.float32),
              jax.ShapeDtypeStruct((NS, EPT), jnp.int32)),
    mesh=_mesh,
    scratch_types=_prep_scratch,
    compiler_params=_sc_params,
)()


def _prop_body(colp_hbm, rowp_hbm, c2_hbm, u2_hbm, s_out, g_hbm,
               col_sb, rowp_sb, gbufa, gbufb, sbuf, ubuf, cbuf, zbuf,
               gsema, gsemb, ssema, ssemb, s_sh):
    c_id = lax.axis_index("c")
    sid = lax.axis_index("s")
    half = c_id * N_PAD
    r0 = sid * NPT

    pltpu.sync_copy(c2_hbm.at[pl.ds(N, UC)], zbuf)

    def init(j, _):
        rs = r0 + j * UC
        pltpu.sync_copy(zbuf, s_sh.at[pl.ds(rs, UC)])
        pltpu.sync_copy(c2_hbm.at[pl.ds(rs, UC)], cbuf)
        pltpu.sync_copy(cbuf, g_hbm.at[pl.ds(half + rs, UC)])
        return 0
    lax.fori_loop(0, NUP, init, 0)
    plsc.subcore_barrier()

    def round_(t, _):
        def superchunk(sb, _):
            pltpu.sync_copy(colp_hbm.at[c_id, sid, pl.ds(sb * SB, SB)],
                            col_sb)
            pltpu.sync_copy(rowp_hbm.at[sid, pl.ds(sb * SB, SB)], rowp_sb)
            pltpu.async_copy(g_hbm.at[col_sb.at[0]], gbufa, gsema)

            def pair(i, _):
                j0 = 2 * i
                pltpu.make_async_copy(
                    g_hbm.at[col_sb.at[j0]], gbufa, gsema).wait()
                pltpu.async_copy(gbufa, s_sh.at[rowp_sb.at[j0]], ssema,
                                 add=True)

                @pl.when(i > 0)
                def _():
                    pltpu.make_async_copy(
                        gbufb, s_sh.at[rowp_sb.at[j0 - 1]], ssemb).wait()
                pltpu.async_copy(g_hbm.at[col_sb.at[j0 + 1]], gbufb, gsemb)

                pltpu.make_async_copy(
                    g_hbm.at[col_sb.at[j0 + 1]], gbufb, gsemb).wait()
                pltpu.async_copy(gbufb, s_sh.at[rowp_sb.at[j0 + 1]], ssemb,
                                 add=True)

                @pl.when(j0 + 2 < SB)
                def _():
                    pltpu.make_async_copy(
                        gbufa, s_sh.at[rowp_sb.at[j0]], ssema).wait()
                    pltpu.async_copy(
                        g_hbm.at[col_sb.at[j0 + 2]], gbufa, gsema)
                return 0
            lax.fori_loop(0, SB // 2, pair, 0)
            pltpu.make_async_copy(
                gbufa, s_sh.at[rowp_sb.at[SB - 2]], ssema).wait()
            pltpu.make_async_copy(
                gbufb, s_sh.at[rowp_sb.at[SB - 1]], ssemb).wait()
            return 0
        lax.fori_loop(0, NSB, superchunk, 0)
        plsc.subcore_barrier()

        @pl.when(t < POWER1 - 1)
        def _():
            def upd(j, _):
                rs = r0 + j * UC
                pltpu.sync_copy(s_sh.at[pl.ds(rs, UC)], sbuf)
                pltpu.sync_copy(u2_hbm.at[pl.ds(rs, UC)], ubuf)
                pltpu.sync_copy(c2_hbm.at[pl.ds(rs, UC)], cbuf)

                def rowop(r, _):
                    for l in range(DW // 16):
                        sl = pl.ds(l * 16, 16)
                        cbuf[r, sl] = ubuf[r, sl] * sbuf[r, sl] + cbuf[r, sl]
                    return 0
                lax.fori_loop(0, UC, rowop, 0)
                pltpu.sync_copy(zbuf, s_sh.at[pl.ds(rs, UC)])
                pltpu.sync_copy(cbuf, g_hbm.at[pl.ds(half + rs, UC)])
                return 0
            lax.fori_loop(0, NUP, upd, 0)
            plsc.subcore_barrier()
        return 0
    lax.fori_loop(0, POWER1, round_, 0)

    def outc(j, _):
        rs = r0 + j * UC
        pltpu.sync_copy(s_sh.at[pl.ds(rs, UC)], sbuf)
        pltpu.sync_copy(sbuf, s_out.at[pl.ds(half + rs, UC)])
        return 0
    lax.fori_loop(0, NUP, outc, 0)


_prop_scratch = [
        pltpu.VMEM((SB, K), jnp.int32),
        pltpu.VMEM((SB, K), jnp.int32),
        pltpu.VMEM((K, DW), jnp.float32),
        pltpu.VMEM((K, DW), jnp.float32),
        pltpu.VMEM((UC, DW), jnp.float32),
        pltpu.VMEM((UC, DW), jnp.float32),
        pltpu.VMEM((UC, DW), jnp.float32),
        pltpu.VMEM((UC, DW), jnp.float32),
        pltpu.SemaphoreType.DMA,
        pltpu.SemaphoreType.DMA,
        pltpu.SemaphoreType.DMA,
        pltpu.SemaphoreType.DMA,
        pltpu.VMEM_SHARED((N_PAD, DW), jnp.float32),
]

_prop = functools.partial(
    pl.kernel, _prop_body,
    out_type=(jax.ShapeDtypeStruct((NC * N_PAD, DW), jnp.float32),
              jax.ShapeDtypeStruct((NC * N_PAD, DW), jnp.float32)),
    mesh=_mesh,
    scratch_types=_prop_scratch,
    compiler_params=_sc_params,
)()


def _lin_body(x_ref, w_ref, b_ref, deg_ref, xx_ref, c_ref, u_ref):
    xx = jnp.dot(x_ref[...], w_ref[...],
                 preferred_element_type=jnp.float32) + b_ref[...]
    d = deg_ref[...]
    dis = jnp.where(d > 0, lax.rsqrt(jnp.where(d > 0, d, 1.0)), 0.0)
    xx_ref[...] = xx
    c_ref[...] = dis * xx
    u_ref[...] = jnp.broadcast_to(dis * dis, u_ref.shape)


def _linear_scale(x, w, b, deg):
    blk = 1000
    grid = (N // blk,)
    return pl.pallas_call(
        _lin_body,
        grid=grid,
        in_specs=[
            pl.BlockSpec((blk, D), lambda i: (i, 0)),
            pl.BlockSpec((D, D), lambda i: (0, 0)),
            pl.BlockSpec((1, D), lambda i: (0, 0)),
            pl.BlockSpec((blk, 1), lambda i: (i, 0)),
        ],
        out_specs=[
            pl.BlockSpec((blk, D), lambda i: (i, 0)),
            pl.BlockSpec((blk, D), lambda i: (i, 0)),
            pl.BlockSpec((blk, D), lambda i: (i, 0)),
        ],
        out_shape=[
            jax.ShapeDtypeStruct((N, D), jnp.float32),
            jax.ShapeDtypeStruct((N, D), jnp.float32),
            jax.ShapeDtypeStruct((N, D), jnp.float32),
        ],
    )(x, w, b, deg)


def _fin_body(s_ref, xx_ref, deg_ref, pp0_ref, pp1_ref, wp_ref, bp_ref,
              out_ref):
    d = deg_ref[...]
    dis = jnp.where(d > 0, lax.rsqrt(jnp.where(d > 0, d, 1.0)), 0.0)
    xx = xx_ref[...]
    hx = dis * s_ref[...] + xx
    h = jnp.maximum(pp0_ref[...] * xx + pp1_ref[...] * hx, 0.0)
    out_ref[...] = jnp.dot(h, wp_ref[...],
                           preferred_element_type=jnp.float32) + bp_ref[...]


def _final(s7, xx, deg, pp0, pp1, wp, bp):
    blk = 1000
    grid = (N // blk,)
    d_out = wp.shape[1]
    return pl.pallas_call(
        _fin_body,
        grid=grid,
        in_specs=[
            pl.BlockSpec((blk, D), lambda i: (i, 0)),
            pl.BlockSpec((blk, D), lambda i: (i, 0)),
            pl.BlockSpec((blk, 1), lambda i: (i, 0)),
            pl.BlockSpec((1, D), lambda i: (0, 0)),
            pl.BlockSpec((1, D), lambda i: (0, 0)),
            pl.BlockSpec((D, d_out), lambda i: (0, 0)),
            pl.BlockSpec((1, d_out), lambda i: (0, 0)),
        ],
        out_specs=pl.BlockSpec((blk, d_out), lambda i: (i, 0)),
        out_shape=jax.ShapeDtypeStruct((N, d_out), jnp.float32),
    )(s7, xx, deg, pp0, pp1, wp, bp)


_DEBUG_STAGE = 0


@jax.jit
def _impl_debug1(x, edge_index, W_linX, b_linX, policy, W_pred, b_pred):
    row = edge_index[0].reshape(NS, EPT)
    col = edge_index[1].reshape(NS, EPT)
    deg_pad, rowp = _prep(row, col)
    deg = deg_pad[:N]
    rowp = rowp.reshape(E)
    dis = jnp.where(deg > 0, deg ** -0.5, 0.0)
    xX = x @ W_linX + b_linX
    c = dis[:, None] * xX
    u = (dis * dis)[:, None]
    g = c
    colf = edge_index[1]
    for t in range(POWER1):
        S = jnp.zeros((N, D), jnp.float32).at[rowp].add(jnp.take(g, colf, axis=0))
        if t < POWER1 - 1:
            g = u * S + c
    hX = dis[:, None] * S + xX
    pp = jax.nn.softmax(policy[:2])
    h = jax.nn.relu(pp[0] * xX + pp[1] * hX)
    return h @ W_pred + b_pred


@jax.jit
def _impl(x, edge_index, W_linX, b_linX, policy, W_pred, b_pred):
    row = edge_index[0].reshape(NS, EPT)
    col = edge_index[1].reshape(NS, EPT)

    deg_pad, rowp = _prep(row, col)

    colp = jnp.concatenate(
        [col, jnp.zeros((NS, EPT_PAD - EPT), jnp.int32)],
        axis=1).reshape(NS, NCH, K)
    colp = jnp.stack([colp, colp + N_PAD])
    rowp_p = jnp.concatenate(
        [rowp, jnp.full((NS, EPT_PAD - EPT), N_PAD - 1, jnp.int32)],
        axis=1).reshape(NS, NCH, K)

    deg2 = deg_pad[:N, None]
    xx, c, uf = _linear_scale(x, W_linX, b_linX.reshape(1, D), deg2)

    c2 = jnp.zeros((N_PAD, DW), jnp.float32).at[:N].set(c)
    u2 = jnp.zeros((N_PAD, DW), jnp.float32).at[:N].set(uf)

    s_halves, _g = _prop(colp, rowp_p, c2, u2)
    s7 = s_halves[:N]

    e = jnp.exp(policy[:2] - jnp.max(policy[:2]))
    pp = e / jnp.sum(e)
    pp0 = jnp.full((1, D), pp[0], jnp.float32)
    pp1 = jnp.full((1, D), pp[1], jnp.float32)

    return _final(s7, xx, deg2, pp0, pp1, W_pred,
                  b_pred.reshape(1, -1))


def kernel(x, edge_index, W_linX, b_linX, policy, W_pred, b_pred):
    if _DEBUG_STAGE == 1:
        return _impl_debug1(x, edge_index, W_linX, b_linX, policy,
                            W_pred, b_pred)
    return _impl(x, edge_index, W_linX, b_linX, policy, W_pred, b_pred)


# speedup vs baseline: 1.3797x; 1.3797x over previous
import functools

import jax
import jax.numpy as jnp
from jax import lax
from jax.experimental import pallas as pl
from jax.experimental.pallas import tpu as pltpu
from jax.experimental.pallas import tpu_sc as plsc

N = 10000
E = 320000
D = 128
DW = 128
N_PAD = 10240
POWER1 = 8
NC = 2
NS = 16
EPT = E // NS
K = 128
SB = 16
NSB = (EPT + SB * K - 1) // (SB * K)
NCH = NSB * SB
EPT_PAD = NCH * K
NPT = N_PAD // NS
UC = 16
NUP = NPT // UC
EPC = E // NC
EPT2 = EPC // NS
NSB2 = (EPT2 + SB * K - 1) // (SB * K)
NCH2 = NSB2 * SB
EPT2_PAD = NCH2 * K
ZR = 64

_mesh = plsc.VectorSubcoreMesh(core_axis_name="c", subcore_axis_name="s")
_sc_params = pltpu.CompilerParams(needs_layout_passes=False)


def _prep_body(row_hbm, col_hbm, deg_out, rowp_out,
               eraw, hist, tmp640, dtmp, estage, sh_hist):
    c_id = lax.axis_index("c")
    sid = lax.axis_index("s")

    @pl.when(c_id == 0)
    def _():
        pltpu.sync_copy(col_hbm.at[sid], eraw)

        def zb(k, _):
            hist[pl.ds(k * 16, 16)] = jnp.zeros((16,), jnp.float32)
            return 0
        lax.fori_loop(0, N_PAD // 16, zb, 0)

        ones = jnp.ones((16,), jnp.float32)

        def hb(k, _):
            idx = eraw[pl.ds(k * 16, 16)]
            plsc.addupdate_scatter(hist, [idx], ones)
            return 0
        lax.fori_loop(0, EPT // 16, hb, 0)

        pltpu.sync_copy(hist, sh_hist.at[sid])
        plsc.subcore_barrier()
        r0 = sid * NPT
        pltpu.sync_copy(sh_hist.at[0, pl.ds(r0, NPT)], dtmp)

        def cb(t, _):
            pltpu.sync_copy(sh_hist.at[t, pl.ds(r0, NPT)], tmp640)

            def ab(k, _):
                dtmp[pl.ds(k * 16, 16)] = (dtmp[pl.ds(k * 16, 16)]
                                           + tmp640[pl.ds(k * 16, 16)])
                return 0
            lax.fori_loop(0, NPT // 16, ab, 0)
            return 0
        lax.fori_loop(1, NS, cb, 0)
        pltpu.sync_copy(dtmp, deg_out.at[pl.ds(r0, NPT)])

    @pl.when(c_id == 1)
    def _():
        def mt(t, m):
            pltpu.sync_copy(row_hbm.at[t], eraw)

            def mb(k, mm):
                return jnp.minimum(mm, eraw[pl.ds(k * 16, 16)])
            return lax.fori_loop(0, EPT // 16, mb, m)
        m = lax.fori_loop(0, NS, mt,
                          jnp.full((16,), 2**31 - 1, jnp.int32))
        rminf = jnp.min(m.astype(jnp.float32))
        rminv = jnp.broadcast_to(rminf, (16,)).astype(jnp.int32)

        pltpu.sync_copy(row_hbm.at[sid], eraw)

        def rb(k, _):
            estage[pl.ds(k * 16, 16)] = eraw[pl.ds(k * 16, 16)] - rminv
            return 0
        lax.fori_loop(0, EPT // 16, rb, 0)
        pltpu.sync_copy(estage, rowp_out.at[sid])


_prep_scratch = [
        pltpu.VMEM((EPT,), jnp.int32),
        pltpu.VMEM((N_PAD,), jnp.float32),
        pltpu.VMEM((NPT,), jnp.float32),
        pltpu.VMEM((NPT,), jnp.float32),
        pltpu.VMEM((EPT,), jnp.int32),
        pltpu.VMEM_SHARED((NS, N_PAD), jnp.float32),
]

_prep = functools.partial(
    pl.kernel, _prep_body,
    out_type=(jax.ShapeDtypeStruct((N_PAD,), jnp.float32),
              jax.ShapeDtypeStruct((NS, EPT), jnp.int32)),
    mesh=_mesh,
    scratch_types=_prep_scratch,
    compiler_params=_sc_params,
)()


def _prop_body(colp_hbm, rowp_hbm, g_hbm, s_out,
               col_sb, rowp_sb, gbufa, gbufb, zbuf,
               gsema, gsemb, ssema, ssemb, s_sh):
    c_id = lax.axis_index("c")
    sid = lax.axis_index("s")
    r0 = sid * NPT

    pltpu.sync_copy(g_hbm.at[pl.ds(N, ZR)], zbuf)

    def zi(j, _):
        pltpu.sync_copy(zbuf, s_sh.at[pl.ds(r0 + j * ZR, ZR)])
        return 0
    lax.fori_loop(0, NPT // ZR, zi, 0)
    plsc.subcore_barrier()

    def superchunk(sb, _):
        pltpu.sync_copy(colp_hbm.at[c_id, sid, pl.ds(sb * SB, SB)], col_sb)
        pltpu.sync_copy(rowp_hbm.at[c_id, sid, pl.ds(sb * SB, SB)], rowp_sb)
        pltpu.async_copy(g_hbm.at[col_sb.at[0]], gbufa, gsema)

        def pair(i, _):
            j0 = 2 * i
            pltpu.make_async_copy(
                g_hbm.at[col_sb.at[j0]], gbufa, gsema).wait()
            pltpu.async_copy(gbufa, s_sh.at[rowp_sb.at[j0]], ssema,
                             add=True)

            @pl.when(i > 0)
            def _():
                pltpu.make_async_copy(
                    gbufb, s_sh.at[rowp_sb.at[j0 - 1]], ssemb).wait()
            pltpu.async_copy(g_hbm.at[col_sb.at[j0 + 1]], gbufb, gsemb)

            pltpu.make_async_copy(
                g_hbm.at[col_sb.at[j0 + 1]], gbufb, gsemb).wait()
            pltpu.async_copy(gbufb, s_sh.at[rowp_sb.at[j0 + 1]], ssemb,
                             add=True)

            @pl.when(j0 + 2 < SB)
            def _():
                pltpu.make_async_copy(
                    gbufa, s_sh.at[rowp_sb.at[j0]], ssema).wait()
                pltpu.async_copy(
                    g_hbm.at[col_sb.at[j0 + 2]], gbufa, gsema)
            return 0
        lax.fori_loop(0, SB // 2, pair, 0)
        pltpu.make_async_copy(
            gbufa, s_sh.at[rowp_sb.at[SB - 2]], ssema).wait()
        pltpu.make_async_copy(
            gbufb, s_sh.at[rowp_sb.at[SB - 1]], ssemb).wait()
        return 0
    lax.fori_loop(0, NSB2, superchunk, 0)
    plsc.subcore_barrier()

    pltpu.sync_copy(s_sh.at[pl.ds(r0, NPT)],
                    s_out.at[pl.ds(c_id * N_PAD + r0, NPT)])


_prop_scratch = [
        pltpu.VMEM((SB, K), jnp.int32),
        pltpu.VMEM((SB, K), jnp.int32),
        pltpu.VMEM((K, DW), jnp.float32),
        pltpu.VMEM((K, DW), jnp.float32),
        pltpu.VMEM((ZR, DW), jnp.float32),
        pltpu.SemaphoreType.DMA,
        pltpu.SemaphoreType.DMA,
        pltpu.SemaphoreType.DMA,
        pltpu.SemaphoreType.DMA,
        pltpu.VMEM_SHARED((N_PAD, DW), jnp.float32),
]

_prop = functools.partial(
    pl.kernel, _prop_body,
    out_type=(jax.ShapeDtypeStruct((NC * N_PAD, DW), jnp.float32),),
    mesh=_mesh,
    scratch_types=_prop_scratch,
    compiler_params=_sc_params,
)()


def _upd_body(s0_ref, s1_ref, u_ref, c_ref, g_ref):
    g_ref[...] = u_ref[...] * (s0_ref[...] + s1_ref[...]) + c_ref[...]


def _gupdate(s0, s1, u2, c2):
    blk = 1024
    grid = (N_PAD // blk,)
    spec = pl.BlockSpec((blk, DW), lambda i: (i, 0))
    return pl.pallas_call(
        _upd_body,
        grid=grid,
        in_specs=[spec, spec, spec, spec],
        out_specs=spec,
        out_shape=jax.ShapeDtypeStruct((N_PAD, DW), jnp.float32),
    )(s0, s1, u2, c2)


def _lin_body(x_ref, w_ref, b_ref, deg_ref, xx_ref, c_ref, u_ref):
    xx = jnp.dot(x_ref[...], w_ref[...],
                 preferred_element_type=jnp.float32) + b_ref[...]
    d = deg_ref[...]
    dis = jnp.where(d > 0, lax.rsqrt(jnp.where(d > 0, d, 1.0)), 0.0)
    xx_ref[...] = xx
    c_ref[...] = dis * xx
    u_ref[...] = jnp.broadcast_to(dis * dis, u_ref.shape)


def _linear_scale(x, w, b, deg):
    blk = 1000
    grid = (N // blk,)
    return pl.pallas_call(
        _lin_body,
        grid=grid,
        in_specs=[
            pl.BlockSpec((blk, D), lambda i: (i, 0)),
            pl.BlockSpec((D, D), lambda i: (0, 0)),
            pl.BlockSpec((1, D), lambda i: (0, 0)),
            pl.BlockSpec((blk, 1), lambda i: (i, 0)),
        ],
        out_specs=[
            pl.BlockSpec((blk, D), lambda i: (i, 0)),
            pl.BlockSpec((blk, D), lambda i: (i, 0)),
            pl.BlockSpec((blk, D), lambda i: (i, 0)),
        ],
        out_shape=[
            jax.ShapeDtypeStruct((N, D), jnp.float32),
            jax.ShapeDtypeStruct((N, D), jnp.float32),
            jax.ShapeDtypeStruct((N, D), jnp.float32),
        ],
    )(x, w, b, deg)


def _fin_body(s0_ref, s1_ref, xx_ref, deg_ref, pp0_ref, pp1_ref, wp_ref,
              bp_ref, out_ref):
    d = deg_ref[...]
    dis = jnp.where(d > 0, lax.rsqrt(jnp.where(d > 0, d, 1.0)), 0.0)
    xx = xx_ref[...]
    hx = dis * (s0_ref[...] + s1_ref[...]) + xx
    h = jnp.maximum(pp0_ref[...] * xx + pp1_ref[...] * hx, 0.0)
    out_ref[...] = jnp.dot(h, wp_ref[...],
                           preferred_element_type=jnp.float32) + bp_ref[...]


def _final(s0, s1, xx, deg, pp0, pp1, wp, bp):
    blk = 1000
    grid = (N // blk,)
    d_out = wp.shape[1]
    return pl.pallas_call(
        _fin_body,
        grid=grid,
        in_specs=[
            pl.BlockSpec((blk, D), lambda i: (i, 0)),
            pl.BlockSpec((blk, D), lambda i: (i, 0)),
            pl.BlockSpec((blk, D), lambda i: (i, 0)),
            pl.BlockSpec((blk, 1), lambda i: (i, 0)),
            pl.BlockSpec((1, D), lambda i: (0, 0)),
            pl.BlockSpec((1, D), lambda i: (0, 0)),
            pl.BlockSpec((D, d_out), lambda i: (0, 0)),
            pl.BlockSpec((1, d_out), lambda i: (0, 0)),
        ],
        out_specs=pl.BlockSpec((blk, d_out), lambda i: (i, 0)),
        out_shape=jax.ShapeDtypeStruct((N, d_out), jnp.float32),
    )(s0, s1, xx, deg, pp0, pp1, wp, bp)


_DEBUG_STAGE = 0


@jax.jit
def _impl_debug1(x, edge_index, W_linX, b_linX, policy, W_pred, b_pred):
    row = edge_index[0].reshape(NS, EPT)
    col = edge_index[1].reshape(NS, EPT)
    deg_pad, rowp = _prep(row, col)
    deg = deg_pad[:N]
    rowp = rowp.reshape(E)
    dis = jnp.where(deg > 0, deg ** -0.5, 0.0)
    xX = x @ W_linX + b_linX
    c = dis[:, None] * xX
    u = (dis * dis)[:, None]
    g = c
    colf = edge_index[1]
    for t in range(POWER1):
        S = jnp.zeros((N, D), jnp.float32).at[rowp].add(jnp.take(g, colf, axis=0))
        if t < POWER1 - 1:
            g = u * S + c
    hX = dis[:, None] * S + xX
    pp = jax.nn.softmax(policy[:2])
    h = jax.nn.relu(pp[0] * xX + pp[1] * hX)
    return h @ W_pred + b_pred


@jax.jit
def _impl(x, edge_index, W_linX, b_linX, policy, W_pred, b_pred):
    row = edge_index[0].reshape(NS, EPT)
    col = edge_index[1].reshape(NS, EPT)

    deg_pad, rowp = _prep(row, col)

    colp = jnp.concatenate(
        [col.reshape(NC, NS, EPT2),
         jnp.zeros((NC, NS, EPT2_PAD - EPT2), jnp.int32)],
        axis=2).reshape(NC, NS, NCH2, K)
    rowp_p = jnp.concatenate(
        [rowp.reshape(NC, NS, EPT2),
         jnp.full((NC, NS, EPT2_PAD - EPT2), N_PAD - 1, jnp.int32)],
        axis=2).reshape(NC, NS, NCH2, K)

    deg2 = deg_pad[:N, None]
    xx, c, uf = _linear_scale(x, W_linX, b_linX.reshape(1, D), deg2)

    c2 = jnp.zeros((N_PAD, DW), jnp.float32).at[:N].set(c)
    u2 = jnp.zeros((N_PAD, DW), jnp.float32).at[:N].set(uf)

    g = c2
    for t in range(POWER1):
        (s_parts,) = _prop(colp, rowp_p, g)
        if t < POWER1 - 1:
            g = _gupdate(s_parts[:N_PAD], s_parts[N_PAD:], u2, c2)
    s0 = s_parts[:N]
    s1 = s_parts[N_PAD:N_PAD + N]

    e = jnp.exp(policy[:2] - jnp.max(policy[:2]))
    pp = e / jnp.sum(e)
    pp0 = jnp.full((1, D), pp[0], jnp.float32)
    pp1 = jnp.full((1, D), pp[1], jnp.float32)

    return _final(s0, s1, xx, deg2, pp0, pp1, W_pred,
                  b_pred.reshape(1, -1))


def kernel(x, edge_index, W_linX, b_linX, policy, W_pred, b_pred):
    if _DEBUG_STAGE == 1:
        return _impl_debug1(x, edge_index, W_linX, b_linX, policy,
                            W_pred, b_pred)
    return _impl(x, edge_index, W_linX, b_linX, policy, W_pred, b_pred)


# speedup vs baseline: 1.3799x; 1.0001x over previous
import functools

import jax
import jax.numpy as jnp
from jax import lax
from jax.experimental import pallas as pl
from jax.experimental.pallas import tpu as pltpu
from jax.experimental.pallas import tpu_sc as plsc

N = 10000
E = 320000
D = 128
DW = 128
N_PAD = 10240
POWER1 = 8
NC = 2
NS = 16
EPT = E // NS
K = 128
SB = 16
NSB = (EPT + SB * K - 1) // (SB * K)
NCH = NSB * SB
EPT_PAD = NCH * K
NPT = N_PAD // NS
UC = 16
NUP = NPT // UC
EPC = E // NC
EPT2 = EPC // NS
NSB2 = (EPT2 + SB * K - 1) // (SB * K)
NCH2 = NSB2 * SB
EPT2_PAD = NCH2 * K
ZR = 64

_mesh = plsc.VectorSubcoreMesh(core_axis_name="c", subcore_axis_name="s")
_sc_params = pltpu.CompilerParams(needs_layout_passes=False)


def _prep_body(row_hbm, col_hbm, deg_out, rowp_out,
               eraw, hist, tmp640, dtmp, estage, sh_hist):
    c_id = lax.axis_index("c")
    sid = lax.axis_index("s")

    @pl.when(c_id == 0)
    def _():
        pltpu.sync_copy(col_hbm.at[sid], eraw)

        def zb(k, _):
            hist[pl.ds(k * 16, 16)] = jnp.zeros((16,), jnp.float32)
            return 0
        lax.fori_loop(0, N_PAD // 16, zb, 0)

        ones = jnp.ones((16,), jnp.float32)

        def hb(k, _):
            idx = eraw[pl.ds(k * 16, 16)]
            plsc.addupdate_scatter(hist, [idx], ones)
            return 0
        lax.fori_loop(0, EPT // 16, hb, 0)

        pltpu.sync_copy(hist, sh_hist.at[sid])
        plsc.subcore_barrier()
        r0 = sid * NPT
        pltpu.sync_copy(sh_hist.at[0, pl.ds(r0, NPT)], dtmp)

        def cb(t, _):
            pltpu.sync_copy(sh_hist.at[t, pl.ds(r0, NPT)], tmp640)

            def ab(k, _):
                dtmp[pl.ds(k * 16, 16)] = (dtmp[pl.ds(k * 16, 16)]
                                           + tmp640[pl.ds(k * 16, 16)])
                return 0
            lax.fori_loop(0, NPT // 16, ab, 0)
            return 0
        lax.fori_loop(1, NS, cb, 0)
        pltpu.sync_copy(dtmp, deg_out.at[pl.ds(r0, NPT)])

    @pl.when(c_id == 1)
    def _():
        def mt(t, m):
            pltpu.sync_copy(row_hbm.at[t], eraw)

            def mb(k, mm):
                return jnp.minimum(mm, eraw[pl.ds(k * 16, 16)])
            return lax.fori_loop(0, EPT // 16, mb, m)
        m = lax.fori_loop(0, NS, mt,
                          jnp.full((16,), 2**31 - 1, jnp.int32))
        rminf = jnp.min(m.astype(jnp.float32))
        rminv = jnp.broadcast_to(rminf, (16,)).astype(jnp.int32)

        pltpu.sync_copy(row_hbm.at[sid], eraw)

        def rb(k, _):
            estage[pl.ds(k * 16, 16)] = eraw[pl.ds(k * 16, 16)] - rminv
            return 0
        lax.fori_loop(0, EPT // 16, rb, 0)
        pltpu.sync_copy(estage, rowp_out.at[sid])


_prep_scratch = [
        pltpu.VMEM((EPT,), jnp.int32),
        pltpu.VMEM((N_PAD,), jnp.float32),
        pltpu.VMEM((NPT,), jnp.float32),
        pltpu.VMEM((NPT,), jnp.float32),
        pltpu.VMEM((EPT,), jnp.int32),
        pltpu.VMEM_SHARED((NS, N_PAD), jnp.float32),
]

_prep = functools.partial(
    pl.kernel, _prep_body,
    out_type=(jax.ShapeDtypeStruct((N_PAD,), jnp.float32),
              jax.ShapeDtypeStruct((NS, EPT), jnp.int32)),
    mesh=_mesh,
    scratch_types=_prep_scratch,
    compiler_params=_sc_params,
)()


def _prop_body(colp_hbm, rowp_hbm, g_hbm, s_out,
               col_sb, rowp_sb, gbufa, gbufb, zbuf,
               gsema, gsemb, ssema, ssemb, s_sh):
    c_id = lax.axis_index("c")
    sid = lax.axis_index("s")
    r0 = sid * NPT

    pltpu.sync_copy(g_hbm.at[pl.ds(N, ZR)], zbuf)

    def zi(j, _):
        pltpu.sync_copy(zbuf, s_sh.at[pl.ds(r0 + j * ZR, ZR)])
        return 0
    lax.fori_loop(0, NPT // ZR, zi, 0)
    plsc.subcore_barrier()

    def superchunk(sb, _):
        pltpu.sync_copy(colp_hbm.at[c_id, sid, pl.ds(sb * SB, SB)], col_sb)
        pltpu.sync_copy(rowp_hbm.at[c_id, sid, pl.ds(sb * SB, SB)], rowp_sb)
        pltpu.async_copy(g_hbm.at[col_sb.at[0]], gbufa, gsema)

        def pair(i, _):
            j0 = 2 * i
            pltpu.make_async_copy(
                g_hbm.at[col_sb.at[j0]], gbufa, gsema).wait()
            pltpu.async_copy(gbufa, s_sh.at[rowp_sb.at[j0]], ssema,
                             add=True)

            @pl.when(i > 0)
            def _():
                pltpu.make_async_copy(
                    gbufb, s_sh.at[rowp_sb.at[j0 - 1]], ssemb).wait()
            pltpu.async_copy(g_hbm.at[col_sb.at[j0 + 1]], gbufb, gsemb)

            pltpu.make_async_copy(
                g_hbm.at[col_sb.at[j0 + 1]], gbufb, gsemb).wait()
            pltpu.async_copy(gbufb, s_sh.at[rowp_sb.at[j0 + 1]], ssemb,
                             add=True)

            @pl.when(j0 + 2 < SB)
            def _():
                pltpu.make_async_copy(
                    gbufa, s_sh.at[rowp_sb.at[j0]], ssema).wait()
                pltpu.async_copy(
                    g_hbm.at[col_sb.at[j0 + 2]], gbufa, gsema)
            return 0
        lax.fori_loop(0, SB // 2, pair, 0)
        pltpu.make_async_copy(
            gbufa, s_sh.at[rowp_sb.at[SB - 2]], ssema).wait()
        pltpu.make_async_copy(
            gbufb, s_sh.at[rowp_sb.at[SB - 1]], ssemb).wait()
        return 0
    lax.fori_loop(0, NSB2, superchunk, 0)
    plsc.subcore_barrier()

    pltpu.sync_copy(s_sh.at[pl.ds(r0, NPT)],
                    s_out.at[pl.ds(c_id * N_PAD + r0, NPT)])


_prop_scratch = [
        pltpu.VMEM((SB, K), jnp.int32),
        pltpu.VMEM((SB, K), jnp.int32),
        pltpu.VMEM((K, DW), jnp.float32),
        pltpu.VMEM((K, DW), jnp.float32),
        pltpu.VMEM((ZR, DW), jnp.float32),
        pltpu.SemaphoreType.DMA,
        pltpu.SemaphoreType.DMA,
        pltpu.SemaphoreType.DMA,
        pltpu.SemaphoreType.DMA,
        pltpu.VMEM_SHARED((N_PAD, DW), jnp.float32),
]

_prop = functools.partial(
    pl.kernel, _prop_body,
    out_type=(jax.ShapeDtypeStruct((NC * N_PAD, DW), jnp.float32),),
    mesh=_mesh,
    scratch_types=_prop_scratch,
    compiler_params=_sc_params,
)()


def _upd_body(s0_ref, s1_ref, u_ref, c_ref, g_ref):
    g_ref[...] = u_ref[...] * (s0_ref[...] + s1_ref[...]) + c_ref[...]


def _gupdate(s0, s1, u2, c2):
    blk = 1024
    grid = (N_PAD // blk,)
    spec = pl.BlockSpec((blk, DW), lambda i: (i, 0))
    return pl.pallas_call(
        _upd_body,
        grid=grid,
        in_specs=[spec, spec, spec, spec],
        out_specs=spec,
        out_shape=jax.ShapeDtypeStruct((N_PAD, DW), jnp.float32),
    )(s0, s1, u2, c2)


def _lin_body(x_ref, w_ref, b_ref, deg_ref, xx_ref, c_ref, u_ref):
    xx = jnp.dot(x_ref[...], w_ref[...],
                 preferred_element_type=jnp.float32) + b_ref[...]
    d = deg_ref[...]
    dis = jnp.where(d > 0, lax.rsqrt(jnp.where(d > 0, d, 1.0)), 0.0)
    xx_ref[...] = xx
    c_ref[...] = dis * xx
    u_ref[...] = jnp.broadcast_to(dis * dis, u_ref.shape)


def _linear_scale(x, w, b, deg):
    blk = 1000
    grid = (N // blk,)
    return pl.pallas_call(
        _lin_body,
        grid=grid,
        in_specs=[
            pl.BlockSpec((blk, D), lambda i: (i, 0)),
            pl.BlockSpec((D, D), lambda i: (0, 0)),
            pl.BlockSpec((1, D), lambda i: (0, 0)),
            pl.BlockSpec((blk, 1), lambda i: (i, 0)),
        ],
        out_specs=[
            pl.BlockSpec((blk, D), lambda i: (i, 0)),
            pl.BlockSpec((blk, D), lambda i: (i, 0)),
            pl.BlockSpec((blk, D), lambda i: (i, 0)),
        ],
        out_shape=[
            jax.ShapeDtypeStruct((N, D), jnp.float32),
            jax.ShapeDtypeStruct((N, D), jnp.float32),
            jax.ShapeDtypeStruct((N, D), jnp.float32),
        ],
    )(x, w, b, deg)


def _fin_body(s0_ref, s1_ref, xx_ref, deg_ref, pp0_ref, pp1_ref, wp_ref,
              bp_ref, out_ref):
    d = deg_ref[...]
    dis = jnp.where(d > 0, lax.rsqrt(jnp.where(d > 0, d, 1.0)), 0.0)
    xx = xx_ref[...]
    hx = dis * (s0_ref[...] + s1_ref[...]) + xx
    h = jnp.maximum(pp0_ref[...] * xx + pp1_ref[...] * hx, 0.0)
    out_ref[...] = jnp.dot(h, wp_ref[...],
                           preferred_element_type=jnp.float32) + bp_ref[...]


def _final(s0, s1, xx, deg, pp0, pp1, wp, bp):
    blk = 1000
    grid = (N // blk,)
    d_out = wp.shape[1]
    return pl.pallas_call(
        _fin_body,
        grid=grid,
        in_specs=[
            pl.BlockSpec((blk, D), lambda i: (i, 0)),
            pl.BlockSpec((blk, D), lambda i: (i, 0)),
            pl.BlockSpec((blk, D), lambda i: (i, 0)),
            pl.BlockSpec((blk, 1), lambda i: (i, 0)),
            pl.BlockSpec((1, D), lambda i: (0, 0)),
            pl.BlockSpec((1, D), lambda i: (0, 0)),
            pl.BlockSpec((D, d_out), lambda i: (0, 0)),
            pl.BlockSpec((1, d_out), lambda i: (0, 0)),
        ],
        out_specs=pl.BlockSpec((blk, d_out), lambda i: (i, 0)),
        out_shape=jax.ShapeDtypeStruct((N, d_out), jnp.float32),
    )(s0, s1, xx, deg, pp0, pp1, wp, bp)


@jax.jit
def _impl(x, edge_index, W_linX, b_linX, policy, W_pred, b_pred):
    row = edge_index[0].reshape(NS, EPT)
    col = edge_index[1].reshape(NS, EPT)

    deg_pad, rowp = _prep(row, col)

    colp = jnp.concatenate(
        [col.reshape(NC, NS, EPT2),
         jnp.zeros((NC, NS, EPT2_PAD - EPT2), jnp.int32)],
        axis=2).reshape(NC, NS, NCH2, K)
    rowp_p = jnp.concatenate(
        [rowp.reshape(NC, NS, EPT2),
         jnp.full((NC, NS, EPT2_PAD - EPT2), N_PAD - 1, jnp.int32)],
        axis=2).reshape(NC, NS, NCH2, K)

    deg2 = deg_pad[:N, None]
    xx, c, uf = _linear_scale(x, W_linX, b_linX.reshape(1, D), deg2)

    c2 = jnp.zeros((N_PAD, DW), jnp.float32).at[:N].set(c)
    u2 = jnp.zeros((N_PAD, DW), jnp.float32).at[:N].set(uf)

    g = c2
    for t in range(POWER1):
        (s_parts,) = _prop(colp, rowp_p, g)
        if t < POWER1 - 1:
            g = _gupdate(s_parts[:N_PAD], s_parts[N_PAD:], u2, c2)
    s0 = s_parts[:N]
    s1 = s_parts[N_PAD:N_PAD + N]

    e = jnp.exp(policy[:2] - jnp.max(policy[:2]))
    pp = e / jnp.sum(e)
    pp0 = jnp.full((1, D), pp[0], jnp.float32)
    pp1 = jnp.full((1, D), pp[1], jnp.float32)

    return _final(s0, s1, xx, deg2, pp0, pp1, W_pred,
                  b_pred.reshape(1, -1))


def kernel(x, edge_index, W_linX, b_linX, policy, W_pred, b_pred):
    return _impl(x, edge_index, W_linX, b_linX, policy, W_pred, b_pred)
